# Initial kernel scaffold; baseline (speedup 1.0000x reference)
import jax
import jax.numpy as jnp
from jax.experimental import pallas as pl


def kernel(x, edge_index, W1, W2, W3, W4, W5, Wfc, bfc):
    raise NotImplementedError("write your pallas kernel here")


# speedup vs baseline: 3.0576x; 3.0576x over previous
import functools
import math

import jax
import jax.numpy as jnp
from jax import lax
from jax.experimental import pallas as pl
from jax.experimental.pallas import tpu as pltpu
from jax.experimental.pallas import tpu_sc as plsc

N = 10000
D = 128
E = 320000
C = 40
ALPHA = 0.9
LAMBDA = 1.0

NC = 2
NS = 16
NW = NC * NS
NPAD = 10112
RPT = NPAD // NS
DUMMY = N

CH = 128
EPT = E // NW
CAP = 80


@functools.cache
def _sc_mesh():
    return plsc.VectorSubcoreMesh(core_axis_name="c", subcore_axis_name="s")


@functools.cache
def _sc_deg_fn():
    @functools.partial(
        pl.kernel,
        out_type=jax.ShapeDtypeStruct((NC, NPAD, D), jnp.float32),
        mesh=_sc_mesh(),
        scratch_types=[
            pltpu.VMEM_SHARED((NPAD, D), jnp.float32),
            pltpu.VMEM((CAP, CH), jnp.int32),
            pltpu.VMEM((CH, D), jnp.float32),
        ],
    )
    def deg_kernel(dstp, ones_h, zeros_h, deg_out, deg_sh, dst_v, ones_v):
        c = lax.axis_index("c")
        s = lax.axis_index("s")
        wid = s * NC + c
        pltpu.sync_copy(zeros_h.at[pl.ds(s * RPT, RPT)],
                        deg_sh.at[pl.ds(s * RPT, RPT)])
        pltpu.sync_copy(dstp.at[wid], dst_v)
        pltpu.sync_copy(ones_h, ones_v)
        plsc.subcore_barrier()

        @pl.loop(0, CAP)
        def _(j):
            pltpu.sync_copy(ones_v, deg_sh.at[dst_v.at[j]], add=True)

        plsc.subcore_barrier()
        pltpu.sync_copy(deg_sh.at[pl.ds(s * RPT, RPT)],
                        deg_out.at[c, pl.ds(s * RPT, RPT)])

    return deg_kernel


@functools.cache
def _sc_layer_fn():
    @functools.partial(
        pl.kernel,
        out_type=jax.ShapeDtypeStruct((NC, NPAD, D), jnp.float32),
        mesh=_sc_mesh(),
        scratch_types=[
            pltpu.VMEM_SHARED((NPAD, D), jnp.float32),
            pltpu.VMEM((CAP, CH), jnp.int32),
            pltpu.VMEM((CAP, CH), jnp.int32),
            pltpu.VMEM((CH, D), jnp.float32),
            pltpu.SemaphoreType.DMA,
        ],
    )
    def layer_kernel(hs, srcp, dstp, zeros_h, m_out,
                     m_sh, src_v, dst_v, buf, sem):
        c = lax.axis_index("c")
        s = lax.axis_index("s")
        wid = s * NC + c
        pltpu.sync_copy(zeros_h.at[pl.ds(s * RPT, RPT)],
                        m_sh.at[pl.ds(s * RPT, RPT)])
        pltpu.sync_copy(srcp.at[wid], src_v)
        pltpu.sync_copy(dstp.at[wid], dst_v)
        plsc.subcore_barrier()

        @pl.loop(0, CAP)
        def _(j):
            pltpu.async_copy(hs.at[src_v.at[j]], buf, sem).wait()
            pltpu.sync_copy(buf, m_sh.at[dst_v.at[j]], add=True)

        plsc.subcore_barrier()
        pltpu.sync_copy(m_sh.at[pl.ds(s * RPT, RPT)],
                        m_out.at[c, pl.ds(s * RPT, RPT)])

    return layer_kernel


BLK = 1000


def _tc_prep(deg2, x):
    def body(deg_ref, x_ref, norm_ref, hs_ref):
        d = deg_ref[0, :, 0:1] + deg_ref[1, :, 0:1]
        nrm = lax.rsqrt(jnp.maximum(d, 1.0))
        nb = jnp.broadcast_to(nrm, (BLK, D))
        norm_ref[...] = nb
        hs_ref[...] = x_ref[...] * nb

    return pl.pallas_call(
        body,
        grid=(N // BLK,),
        in_specs=[
            pl.BlockSpec((NC, BLK, D), lambda j: (0, j, 0)),
            pl.BlockSpec((BLK, D), lambda j: (j, 0)),
        ],
        out_specs=[
            pl.BlockSpec((BLK, D), lambda j: (j, 0)),
            pl.BlockSpec((BLK, D), lambda j: (j, 0)),
        ],
        out_shape=[
            jax.ShapeDtypeStruct((N, D), jnp.float32),
            jax.ShapeDtypeStruct((NPAD, D), jnp.float32),
        ],
    )(deg2, x)


def _tc_layer(m2, x, normb, W, beta):
    def body(m_ref, x_ref, n_ref, w_ref, hs_ref):
        mcat = m_ref[0] + m_ref[1]
        nb = n_ref[...]
        g = mcat * nb * (1.0 - ALPHA) + ALPHA * x_ref[...]
        hw = jnp.dot(g, w_ref[...], preferred_element_type=jnp.float32)
        h = jnp.maximum((1.0 - beta) * g + beta * hw, 0.0)
        hs_ref[...] = h * nb

    return pl.pallas_call(
        body,
        grid=(N // BLK,),
        in_specs=[
            pl.BlockSpec((NC, BLK, D), lambda j: (0, j, 0)),
            pl.BlockSpec((BLK, D), lambda j: (j, 0)),
            pl.BlockSpec((BLK, D), lambda j: (j, 0)),
            pl.BlockSpec((D, D), lambda j: (0, 0)),
        ],
        out_specs=pl.BlockSpec((BLK, D), lambda j: (j, 0)),
        out_shape=jax.ShapeDtypeStruct((NPAD, D), jnp.float32),
    )(m2, x, normb, W)


def _tc_final(m2, x, normb, W, Wfc, bfc2, beta):
    def body(m_ref, x_ref, n_ref, w_ref, wfc_ref, b_ref, out_ref):
        mcat = m_ref[0] + m_ref[1]
        nb = n_ref[...]
        g = mcat * nb * (1.0 - ALPHA) + ALPHA * x_ref[...]
        hw = jnp.dot(g, w_ref[...], preferred_element_type=jnp.float32)
        h = jnp.maximum((1.0 - beta) * g + beta * hw, 0.0)
        out_ref[...] = (jnp.dot(h, wfc_ref[...],
                                preferred_element_type=jnp.float32)
                        + b_ref[...])

    return pl.pallas_call(
        body,
        grid=(N // BLK,),
        in_specs=[
            pl.BlockSpec((NC, BLK, D), lambda j: (0, j, 0)),
            pl.BlockSpec((BLK, D), lambda j: (j, 0)),
            pl.BlockSpec((BLK, D), lambda j: (j, 0)),
            pl.BlockSpec((D, D), lambda j: (0, 0)),
            pl.BlockSpec((D, C), lambda j: (0, 0)),
            pl.BlockSpec((1, C), lambda j: (0, 0)),
        ],
        out_specs=pl.BlockSpec((BLK, C), lambda j: (j, 0)),
        out_shape=jax.ShapeDtypeStruct((N, C), jnp.float32),
    )(m2, x, normb, W, Wfc, bfc2)


def kernel(x, edge_index, W1, W2, W3, W4, W5, Wfc, bfc):
    src = edge_index[0].astype(jnp.int32)
    dst = edge_index[1].astype(jnp.int32)
    srcp = jnp.pad(src.reshape(NW, EPT), ((0, 0), (0, CAP * CH - EPT)),
                   constant_values=DUMMY).reshape(NW, CAP, CH)
    dstp = jnp.pad(dst.reshape(NW, EPT), ((0, 0), (0, CAP * CH - EPT)),
                   constant_values=DUMMY).reshape(NW, CAP, CH)
    zerosd = jnp.zeros((NPAD, D), jnp.float32)
    onesd = jnp.ones((CH, D), jnp.float32)

    deg2 = _sc_deg_fn()(dstp, onesd, zerosd)
    normb, hs = _tc_prep(deg2, x)

    Ws = (W1, W2, W3, W4, W5)
    for i in range(4):
        beta = math.log(LAMBDA / (i + 1) + 1.0)
        m2 = _sc_layer_fn()(hs, srcp, dstp, zerosd)
        hs = _tc_layer(m2, x, normb, Ws[i], beta)
    beta = math.log(LAMBDA / 5.0 + 1.0)
    m2 = _sc_layer_fn()(hs, srcp, dstp, zerosd)
    return _tc_final(m2, x, normb, Ws[4], Wfc, bfc.reshape(1, C), beta)
